# Initial kernel scaffold; baseline (speedup 1.0000x reference)
import jax
import jax.numpy as jnp
from jax.experimental import pallas as pl


def kernel(label, dsd_1, dsd_2, usu_1, symp_table, dise_table, W_usu1, W_dsd_2_1, W_dsd_2_2, W_dsd_1_1, W_dsd_1_2):
    raise NotImplementedError("write your pallas kernel here")


# speedup vs baseline: 5.0852x; 5.0852x over previous
import functools

import jax
import jax.numpy as jnp
from jax import lax
from jax.experimental import pallas as pl
from jax.experimental.pallas import tpu as pltpu
from jax.experimental.pallas import tpu_sc as plsc

NUM_SYMP = 100000
NUM_DISE = 1000
D = 64
B = 1024
N1 = 20
N2 = 20
HIST = 200

NC, NS = 2, 16
NW = NC * NS
SPW = B // NW
IC = 104

BB = 128
GB = B // BB
RB = BB * N1


def _sc_body(symp, dise, usu_i, d1_i, d2_i, lbl,
             usu_o, embs_o, sds_o, tgt_o,
             iu, i1, i2, il, ru, r1, r2, rt, su, ss, sem):
  cid = lax.axis_index("c")
  sid = lax.axis_index("s")
  wid = sid * NC + cid
  base = wid * SPW

  pltpu.sync_copy(lbl.at[pl.ds(base, SPW)], il)
  pltpu.async_copy(dise.at[il], rt, sem).wait()
  pltpu.sync_copy(rt, tgt_o.at[pl.ds(base, SPW)])

  zero = jnp.zeros((16,), jnp.float32)

  def sample(t, carry):
    b = base + t
    pltpu.sync_copy(usu_i.at[pl.ds(b * 2, 2)], iu)
    pltpu.sync_copy(d1_i.at[b], i1)
    pltpu.sync_copy(d2_i.at[pl.ds(b * 4, 4)], i2)
    cps = []
    for h in range(2):
      cps.append(pltpu.async_copy(symp.at[iu.at[h]], ru.at[h], sem))
    cps.append(pltpu.async_copy(symp.at[i1], r1, sem))
    for q in range(4):
      cps.append(pltpu.async_copy(dise.at[i2.at[q]], r2.at[q], sem))
    for c in cps:
      c.wait()

    acc = (zero, zero, zero, zero)
    for h in range(2):
      def ubody(r, a, h=h):
        return tuple(a[c] + ru[h, r, pl.ds(c * 16, 16)] for c in range(4))
      acc = lax.fori_loop(0, IC, ubody, acc)
    for c in range(4):
      su[pl.ds(c * 16, 16)] = acc[c]
    pltpu.sync_copy(su, usu_o.at[b])

    pltpu.sync_copy(r1.at[pl.ds(0, N1)], embs_o.at[b])

    for q in range(4):
      for s5 in range(5):
        def dbody(r, a, q=q, s5=s5):
          return tuple(a[c] + r2[q, s5 * 20 + r, pl.ds(c * 16, 16)]
                       for c in range(4))
        a = lax.fori_loop(0, N2, dbody, (zero, zero, zero, zero))
        for c in range(4):
          ss[q * 5 + s5, pl.ds(c * 16, 16)] = a[c]
    pltpu.sync_copy(ss, sds_o.at[b])
    return carry

  lax.fori_loop(0, SPW, sample, 0)


@functools.cache
def _get_sc_gather():
  return pl.kernel(
    _sc_body,
    out_type=[
        jax.ShapeDtypeStruct((B, D), jnp.float32),
        jax.ShapeDtypeStruct((B, N1, D), jnp.float32),
        jax.ShapeDtypeStruct((B, N1, D), jnp.float32),
        jax.ShapeDtypeStruct((B, D), jnp.float32),
    ],
    mesh=plsc.VectorSubcoreMesh(core_axis_name="c", subcore_axis_name="s",
                                num_cores=NC, num_subcores=NS),
    scratch_types=[
        pltpu.VMEM((2, IC), jnp.int32),
        pltpu.VMEM((24,), jnp.int32),
        pltpu.VMEM((4, IC), jnp.int32),
        pltpu.VMEM((SPW,), jnp.int32),
        pltpu.VMEM((2, IC, D), jnp.float32),
        pltpu.VMEM((24, D), jnp.float32),
        pltpu.VMEM((4, IC, D), jnp.float32),
        pltpu.VMEM((SPW, D), jnp.float32),
        pltpu.VMEM((D,), jnp.float32),
        pltpu.VMEM((N1, D), jnp.float32),
        pltpu.SemaphoreType.DMA,
    ],
    compiler_params=pltpu.CompilerParams(use_tc_tiling_on_sc=False),
  )


def _inv_cnt(cnt):
  w = 1.0 / (cnt + 1e-8)
  return jnp.where(w >= 1e8, 0.0, w)


def _tc_body(d1, d2, uu, usum, embs, sds, tgt,
             w_u, w21, w22, w11, w12, out):
  c2 = jnp.sum((d2[...] != 0).astype(jnp.float32), axis=1, keepdims=True)
  sd_avg = sds[...] * _inv_cnt(c2)
  es = embs[...]
  f32 = jnp.float32
  t = (jnp.dot(es + sd_avg, w21[...], preferred_element_type=f32)
       + jnp.dot(sd_avg * es, w22[...], preferred_element_type=f32))
  h = jnp.tanh(t)
  nrm = jnp.sqrt(jnp.sum(h * h, axis=1, keepdims=True))
  e1 = h / jnp.maximum(nrm, 1e-12)

  tg = tgt[...]
  tg_rep = jnp.broadcast_to(tg[:, None, :], (BB, N1, D)).reshape(RB, D)
  msg = (jnp.dot(e1, w11[...], preferred_element_type=f32)
         + jnp.dot(e1 * tg_rep, w12[...], preferred_element_type=f32))
  pooled = jnp.sum(msg.reshape(BB, N1, D), axis=1)
  c1 = jnp.sum((d1[...] != 0).astype(jnp.float32), axis=1, keepdims=True)
  emb_dise = jnp.tanh(pooled * _inv_cnt(c1)
                      + jnp.dot(tg, w11[...], preferred_element_type=f32))

  cu = jnp.sum((uu[...] != 0).astype(jnp.float32), axis=1, keepdims=True)
  emb_user = jnp.tanh(jnp.dot(usum[...] * _inv_cnt(cu), w_u[...],
                              preferred_element_type=f32))
  out[...] = jnp.sum(emb_dise * emb_user, axis=1)


def _tc_dense(d1, d2r, uu, usum, embs, sds, tgt, w_u, w21, w22, w11, w12,
              interpret=False):
  wspec = pl.BlockSpec((D, D), lambda i: (0, 0))
  return pl.pallas_call(
      _tc_body,
      grid=(GB,),
      in_specs=[
          pl.BlockSpec((BB, N1), lambda i: (i, 0)),
          pl.BlockSpec((RB, N2), lambda i: (i, 0)),
          pl.BlockSpec((BB, HIST), lambda i: (i, 0)),
          pl.BlockSpec((BB, D), lambda i: (i, 0)),
          pl.BlockSpec((RB, D), lambda i: (i, 0)),
          pl.BlockSpec((RB, D), lambda i: (i, 0)),
          pl.BlockSpec((BB, D), lambda i: (i, 0)),
          wspec, wspec, wspec, wspec, wspec,
      ],
      out_specs=pl.BlockSpec((BB,), lambda i: (i,)),
      out_shape=jax.ShapeDtypeStruct((B,), jnp.float32),
      interpret=interpret,
  )(d1, d2r, uu, usum, embs, sds, tgt, w_u, w21, w22, w11, w12)


@jax.jit
def kernel(label, dsd_1, dsd_2, usu_1, symp_table, dise_table,
           W_usu1, W_dsd_2_1, W_dsd_2_2, W_dsd_1_1, W_dsd_1_2):
  label = label.astype(jnp.int32)
  dsd_1 = dsd_1.astype(jnp.int32)
  dsd_2 = dsd_2.astype(jnp.int32)
  usu_1 = usu_1.astype(jnp.int32)

  usu_p = jnp.pad(usu_1.reshape(B, 2, HIST // 2),
                  ((0, 0), (0, 0), (0, IC - HIST // 2))).reshape(B * 2, IC)
  d1_p = jnp.pad(dsd_1, ((0, 0), (0, 4)))
  d2_p = jnp.pad(dsd_2.reshape(B, 4, 100),
                 ((0, 0), (0, 0), (0, IC - 100))).reshape(B * 4, IC)

  usum, embs, sds, tgt = _get_sc_gather()(symp_table, dise_table,
                                          usu_p, d1_p, d2_p, label)
  score = _tc_dense(dsd_1, dsd_2.reshape(B * N1, N2), usu_1, usum,
                    embs.reshape(B * N1, D), sds.reshape(B * N1, D), tgt,
                    W_usu1, W_dsd_2_1, W_dsd_2_2, W_dsd_1_1, W_dsd_1_2)
  return score
